# baseline (speedup 1.0000x reference)
import functools

import jax
import jax.numpy as jnp
from jax import lax
from jax.experimental import pallas as pl
from jax.experimental.pallas import tpu as pltpu
from jax.experimental.pallas import tpu_sc as plsc

NUM_EMB = 100000
DIM = 300
BATCH = 16384

_info = plsc.get_sparse_core_info()
_NC, _NS, _L = _info.num_cores, _info.num_subcores, _info.num_lanes
_NW = _NC * _NS
_TPW = (DIM + _NW - 1) // _NW
_OCH = 4096
_NCH = BATCH // _OCH


def _gather_body(idx_hbm, tT_hbm, outT_hbm, idx_v, row_v, oc0, oc1, so0, so1, sr):
    wid = lax.axis_index("s") * _NC + lax.axis_index("c")

    pltpu.sync_copy(idx_hbm, idx_v)

    oc = (oc0, oc1)
    so = (so0, so1)

    def drain(b):
        pltpu.make_async_copy(
            oc[b], outT_hbm.at[0, pl.ds(0, _OCH)], so[b]
        ).wait()

    for b in range(2):
        pltpu.async_copy(oc[b], outT_hbm.at[wid, pl.ds(b * _OCH, _OCH)], so[b])

    def row_iter(t):
        j = wid + _NW * t

        @pl.when(j < DIM)
        def _():
            h = pltpu.async_copy(tT_hbm.at[j], row_v, sr)
            drain(0)
            drain(1)
            h.wait()

            for m in range(_NCH):
                b = m % 2
                c0 = m * _OCH
                if m >= 2:
                    drain(b)
                for u in range(0, _OCH // _L, 16):
                    idxs = [
                        idx_v[pl.ds(c0 + (u + v) * _L, _L)] for v in range(16)
                    ]
                    vals = [plsc.load_gather(row_v, [iv]) for iv in idxs]
                    for v in range(16):
                        oc[b][pl.ds((u + v) * _L, _L)] = vals[v]
                pltpu.async_copy(oc[b], outT_hbm.at[j, pl.ds(c0, _OCH)], so[b])

    pl.loop(0, _TPW)(row_iter)

    drain(0)
    drain(1)


@jax.jit
def kernel(label_idx, table):
    mesh = plsc.VectorSubcoreMesh(core_axis_name="c", subcore_axis_name="s")
    k = functools.partial(
        pl.kernel,
        mesh=mesh,
        out_type=jax.ShapeDtypeStruct((DIM, BATCH), jnp.float32),
        scratch_types=[
            pltpu.VMEM((BATCH,), jnp.int32),
            pltpu.VMEM((NUM_EMB,), jnp.float32),
            pltpu.VMEM((_OCH,), jnp.float32),
            pltpu.VMEM((_OCH,), jnp.float32),
            pltpu.SemaphoreType.DMA,
            pltpu.SemaphoreType.DMA,
            pltpu.SemaphoreType.DMA,
        ],
        compiler_params=pltpu.CompilerParams(needs_layout_passes=False),
    )(_gather_body)
    return k(label_idx, table.T).T


# speedup vs baseline: 1.0056x; 1.0056x over previous
import functools

import jax
import jax.numpy as jnp
from jax import lax
from jax.experimental import pallas as pl
from jax.experimental.pallas import tpu as pltpu
from jax.experimental.pallas import tpu_sc as plsc

NUM_EMB = 100000
DIM = 300
BATCH = 16384

_info = plsc.get_sparse_core_info()
_NC, _NS, _L = _info.num_cores, _info.num_subcores, _info.num_lanes
_NW = _NC * _NS
_TPW = (DIM + _NW - 1) // _NW
_OCH = 4096
_NCH = BATCH // _OCH


def _gather_body(idx_hbm, tT_hbm, outT_hbm, idx_v, row_v, oc0, oc1, so0, so1, sr):
    wid = lax.axis_index("s") * _NC + lax.axis_index("c")

    pltpu.sync_copy(idx_hbm, idx_v)

    oc = (oc0, oc1)
    so = (so0, so1)

    def drain(b):
        pltpu.make_async_copy(
            oc[b], outT_hbm.at[0, pl.ds(0, _OCH)], so[b]
        ).wait()

    for b in range(2):
        pltpu.async_copy(oc[b], outT_hbm.at[wid, pl.ds(b * _OCH, _OCH)], so[b])

    def row_iter(t):
        j = wid + _NW * t

        @pl.when(j < DIM)
        def _():
            h = pltpu.async_copy(tT_hbm.at[j], row_v, sr)
            drain(0)
            drain(1)
            h.wait()

            for m in range(_NCH):
                b = m % 2
                c0 = m * _OCH
                if m >= 2:
                    drain(b)
                for u in range(0, _OCH // _L, 8):
                    idxs = [
                        idx_v[pl.ds(c0 + (u + v) * _L, _L)] for v in range(8)
                    ]
                    vals = [plsc.load_gather(row_v, [iv]) for iv in idxs]
                    for v in range(8):
                        oc[b][pl.ds((u + v) * _L, _L)] = vals[v]
                pltpu.async_copy(oc[b], outT_hbm.at[j, pl.ds(c0, _OCH)], so[b])

    pl.loop(0, _TPW)(row_iter)

    drain(0)
    drain(1)


@jax.jit
def kernel(label_idx, table):
    mesh = plsc.VectorSubcoreMesh(core_axis_name="c", subcore_axis_name="s")
    k = functools.partial(
        pl.kernel,
        mesh=mesh,
        out_type=jax.ShapeDtypeStruct((DIM, BATCH), jnp.float32),
        scratch_types=[
            pltpu.VMEM((BATCH,), jnp.int32),
            pltpu.VMEM((NUM_EMB,), jnp.float32),
            pltpu.VMEM((_OCH,), jnp.float32),
            pltpu.VMEM((_OCH,), jnp.float32),
            pltpu.SemaphoreType.DMA,
            pltpu.SemaphoreType.DMA,
            pltpu.SemaphoreType.DMA,
        ],
        compiler_params=pltpu.CompilerParams(needs_layout_passes=False),
    )(_gather_body)
    return k(label_idx, table.T).T


# speedup vs baseline: 1.0097x; 1.0041x over previous
import functools

import jax
import jax.numpy as jnp
from jax import lax
from jax.experimental import pallas as pl
from jax.experimental.pallas import tpu as pltpu
from jax.experimental.pallas import tpu_sc as plsc

NUM_EMB = 100000
DIM = 300
BATCH = 16384

_info = plsc.get_sparse_core_info()
_NC, _NS, _L = _info.num_cores, _info.num_subcores, _info.num_lanes
_NW = _NC * _NS
_TPW = (DIM + _NW - 1) // _NW
_OCH = 4096
_NCH = BATCH // _OCH


def _gather_body(idx_hbm, tT_hbm, outT_hbm, idx_v, row_v, oc0, oc1, so0, so1):
    wid = lax.axis_index("s") * _NC + lax.axis_index("c")

    pltpu.sync_copy(idx_hbm, idx_v)

    oc = (oc0, oc1)
    so = (so0, so1)

    def drain(b):
        pltpu.make_async_copy(
            oc[b], outT_hbm.at[0, pl.ds(0, _OCH)], so[b]
        ).wait()

    for b in range(2):
        pltpu.async_copy(oc[b], outT_hbm.at[wid, pl.ds(b * _OCH, _OCH)], so[b])

    def row_iter(t):
        j = wid + _NW * t

        @pl.when(j < DIM)
        def _():
            pltpu.sync_copy(tT_hbm.at[j], row_v)

            for m in range(_NCH):
                b = m % 2
                c0 = m * _OCH
                drain(b)
                for u in range(0, _OCH // _L, 8):
                    idxs = [
                        idx_v[pl.ds(c0 + (u + v) * _L, _L)] for v in range(8)
                    ]
                    vals = [plsc.load_gather(row_v, [iv]) for iv in idxs]
                    for v in range(8):
                        oc[b][pl.ds((u + v) * _L, _L)] = vals[v]
                pltpu.async_copy(oc[b], outT_hbm.at[j, pl.ds(c0, _OCH)], so[b])

    pl.loop(0, _TPW)(row_iter)

    drain(0)
    drain(1)


@jax.jit
def kernel(label_idx, table):
    mesh = plsc.VectorSubcoreMesh(core_axis_name="c", subcore_axis_name="s")
    k = functools.partial(
        pl.kernel,
        mesh=mesh,
        out_type=jax.ShapeDtypeStruct((DIM, BATCH), jnp.float32),
        scratch_types=[
            pltpu.VMEM((BATCH,), jnp.int32),
            pltpu.VMEM((NUM_EMB,), jnp.float32),
            pltpu.VMEM((_OCH,), jnp.float32),
            pltpu.VMEM((_OCH,), jnp.float32),
            pltpu.SemaphoreType.DMA,
            pltpu.SemaphoreType.DMA,
        ],
        compiler_params=pltpu.CompilerParams(needs_layout_passes=False),
    )(_gather_body)
    return k(label_idx, table.T).T
